# baseline (speedup 1.0000x reference)
import jax
import jax.numpy as jnp
from jax import lax
from jax.experimental import pallas as pl
from jax.experimental.pallas import tpu as pltpu
from jax.experimental.pallas import tpu_sc as plsc

N = 10000
E = 320000
D = 128

NC = 2
NS = 16
NW = NC * NS
CH = 50
NCHUNK = 200
WC = 40
NWIN = NCHUNK // WC
NBUF = 4
NA = N
FR = 624
CR = 208
TAILF = N - NS * FR
TAILZ = NA - NS * FR


def _make_sc_agg(with_deg):
  mesh = plsc.VectorSubcoreMesh(core_axis_name="c", subcore_axis_name="s")
  out_type = [jax.ShapeDtypeStruct((NC, N, D), jnp.float32)]
  if with_deg:
    out_type.append(jax.ShapeDtypeStruct((NC, NA), jnp.float32))

  scratch = (
      [pltpu.VMEM_SHARED((NA, D), jnp.float32)]
      + [pltpu.VMEM((WC, CH), jnp.int32)] * 2
      + [pltpu.VMEM((CH, D), jnp.float32)] * NBUF
      + [pltpu.SemaphoreType.DMA] * NBUF
      + [pltpu.SemaphoreType.DMA] * NBUF
  )
  if with_deg:
    scratch += [
        pltpu.VMEM_SHARED((NA,), jnp.float32),
        pltpu.VMEM((128,), jnp.float32),
        pltpu.VMEM((FR,), jnp.float32),
        pltpu.SemaphoreType.DMA,
    ]

  def body(x_hbm, src_hbm, dst_hbm, z2d_hbm, *rest):
    if with_deg:
      sums_hbm, deg_hbm = rest[0], rest[1]
      rest = rest[2:]
    else:
      sums_hbm = rest[0]
      rest = rest[1:]
    acc, src_v, dst_v = rest[0], rest[1], rest[2]
    rows = rest[3:3 + NBUF]
    gsem = rest[3 + NBUF:3 + 2 * NBUF]
    ssem = rest[3 + 2 * NBUF:3 + 3 * NBUF]
    if with_deg:
      dacc, ones, dz, dsem = rest[3 + 3 * NBUF:]
    c = lax.axis_index("c")
    s = lax.axis_index("s")
    wid = c * NS + s

    for t in range(FR // CR):
      r0 = s * FR + t * CR
      pltpu.sync_copy(z2d_hbm.at[pl.ds(r0, CR), :], acc.at[pl.ds(r0, CR), :])
    @pl.when(s == NS - 1)
    def _():
      r0 = NS * FR
      pltpu.sync_copy(z2d_hbm.at[pl.ds(r0, TAILZ), :],
                      acc.at[pl.ds(r0, TAILZ), :])
    if with_deg:
      def zbody(i, _):
        dz[pl.ds(i * 16, 16)] = jnp.zeros((16,), jnp.float32)
        return 0
      lax.fori_loop(0, FR // 16, zbody, 0)
      pltpu.sync_copy(dz, dacc.at[pl.ds(s * FR, FR)])
      @pl.when(s == NS - 1)
      def _():
        pltpu.sync_copy(dz.at[pl.ds(0, TAILZ)],
                        dacc.at[pl.ds(NS * FR, TAILZ)])
      def obody(i, _):
        ones[pl.ds(i * 16, 16)] = jnp.ones((16,), jnp.float32)
        return 0
      lax.fori_loop(0, 8, obody, 0)
    plsc.subcore_barrier()

    def fire_gather(j, b):
      pltpu.async_copy(x_hbm.at[src_v.at[j]], rows[b], gsem[b])

    def wait_gather(j, b):
      pltpu.make_async_copy(x_hbm.at[src_v.at[j]], rows[b], gsem[b]).wait()

    def fire_scatter(j, b):
      pltpu.async_copy(rows[b], acc.at[dst_v.at[j]], ssem[b], add=True)

    def wait_scatter(b):
      pltpu.make_async_copy(rows[b], acc.at[dst_v.at[0]], ssem[b]).wait()

    for win in range(NWIN):
      pltpu.sync_copy(src_hbm.at[wid, pl.ds(win * WC, WC)], src_v)
      pltpu.sync_copy(dst_hbm.at[wid, pl.ds(win * WC, WC)], dst_v)
      fire_gather(0, 0)
      fire_gather(1, 1)

      def quad(p, _):
        for b in range(NBUF):
          j = NBUF * p + b
          wait_gather(j, b)
          @pl.when(j >= 2)
          def _():
            wait_scatter((b - 2) % NBUF)
          @pl.when(j + 2 < WC)
          def _():
            fire_gather(j + 2, (b + 2) % NBUF)
          fire_scatter(j, b)
          if with_deg:
            @pl.when(j >= 1)
            def _():
              pltpu.make_async_copy(ones.at[pl.ds(0, CH)],
                                    dacc.at[dst_v.at[0]], dsem).wait()
            pltpu.async_copy(ones.at[pl.ds(0, CH)], dacc.at[dst_v.at[j]],
                             dsem, add=True)
        return 0
      lax.fori_loop(0, WC // NBUF, quad, 0)

      wait_scatter((WC - 2) % NBUF)
      wait_scatter((WC - 1) % NBUF)
      if with_deg:
        pltpu.make_async_copy(ones.at[pl.ds(0, CH)], dacc.at[dst_v.at[0]],
                              dsem).wait()

    plsc.subcore_barrier()

    for t in range(FR // CR):
      r0 = s * FR + t * CR
      pltpu.sync_copy(acc.at[pl.ds(r0, CR), :], sums_hbm.at[c, pl.ds(r0, CR), :])
    @pl.when(s == NS - 1)
    def _():
      r0 = NS * FR
      pltpu.sync_copy(acc.at[pl.ds(r0, TAILF), :],
                      sums_hbm.at[c, pl.ds(r0, TAILF), :])
    if with_deg:
      @pl.when(s == 0)
      def _():
        pltpu.sync_copy(dacc, deg_hbm.at[c])

  return pl.kernel(body, out_type=out_type, mesh=mesh, scratch_types=scratch)


_sc_agg_deg = _make_sc_agg(True)
_sc_agg = _make_sc_agg(False)


BN = 1000


def _tc1_body(x_ref, s0_ref, s1_ref, d0_ref, d1_ref, w_ref, b_ref, o_ref):
  deg = jnp.maximum(d0_ref[...] + d1_ref[...], 1.0)
  agg = (s0_ref[...] + s1_ref[...]) / deg
  h = (x_ref[...] + agg) @ w_ref[...] + b_ref[...]
  o_ref[...] = jnp.maximum(h, 0.0)


def _tc1(x, s0, s1, d0, d1, w, b):
  row = pl.BlockSpec((BN, D), lambda i: (i, 0))
  col = pl.BlockSpec((BN, 1), lambda i: (i, 0))
  full = pl.BlockSpec((D, D), lambda i: (0, 0))
  bias = pl.BlockSpec((1, D), lambda i: (0, 0))
  return pl.pallas_call(
      _tc1_body,
      grid=(N // BN,),
      in_specs=[row, row, row, col, col, full, bias],
      out_specs=row,
      out_shape=jax.ShapeDtypeStruct((N, D), jnp.float32),
  )(x, s0, s1, d0, d1, w, b)


def _tc2_body(h_ref, s0_ref, s1_ref, d0_ref, d1_ref, w2_ref, b2_ref,
              wd1_ref, bd1_ref, wd2_ref, bd2_ref, hg_ref, rec_ref, acc_ref):
  i = pl.program_id(0)

  @pl.when(i == 0)
  def _():
    acc_ref[...] = jnp.zeros_like(acc_ref)

  deg = jnp.maximum(d0_ref[...] + d1_ref[...], 1.0)
  agg = (s0_ref[...] + s1_ref[...]) / deg
  h2 = jnp.maximum((h_ref[...] + agg) @ w2_ref[...] + b2_ref[...], 0.0)
  acc_ref[...] += jnp.sum(h2, axis=0, keepdims=True)

  @pl.when(i == pl.num_programs(0) - 1)
  def _():
    hg = acc_ref[...] * (1.0 / N)
    hg_ref[...] = hg
    r = jnp.maximum(hg @ wd1_ref[...] + bd1_ref[...], 0.0)
    rec_ref[...] = r @ wd2_ref[...] + bd2_ref[...]


def _tc2(h, s0, s1, d0, d1, w2, b2, wd1, bd1, wd2, bd2):
  row = pl.BlockSpec((BN, D), lambda i: (i, 0))
  col = pl.BlockSpec((BN, 1), lambda i: (i, 0))
  full = pl.BlockSpec((D, D), lambda i: (0, 0))
  bias = pl.BlockSpec((1, D), lambda i: (0, 0))
  out = pl.BlockSpec((1, D), lambda i: (0, 0))
  return pl.pallas_call(
      _tc2_body,
      grid=(N // BN,),
      in_specs=[row, row, row, col, col, full, bias, full, bias, full, bias],
      out_specs=[out, out],
      out_shape=[jax.ShapeDtypeStruct((1, D), jnp.float32),
                 jax.ShapeDtypeStruct((1, D), jnp.float32)],
      scratch_shapes=[pltpu.VMEM((1, D), jnp.float32)],
  )(h, s0, s1, d0, d1, w2, b2, wd1, bd1, wd2, bd2)


@jax.jit
def kernel(features, edge_index, W1, b1, W2, b2, Wd1, bd1, Wd2, bd2):
  src = edge_index[0].reshape(NW, NCHUNK, CH)
  dst = edge_index[1].reshape(NW, NCHUNK, CH)
  z2d = jnp.zeros((NA, D), jnp.float32)

  sums1, deg = _sc_agg_deg(features, src, dst, z2d)
  d0 = deg[0, :N].reshape(N, 1)
  d1 = deg[1, :N].reshape(N, 1)
  h1 = _tc1(features, sums1[0], sums1[1], d0, d1, W1, b1.reshape(1, D))

  (sums2,) = _sc_agg(h1, src, dst, z2d)
  hg, rec = _tc2(h1, sums2[0], sums2[1], d0, d1, W2, b2.reshape(1, D),
                 Wd1, bd1.reshape(1, D), Wd2, bd2.reshape(1, D))
  return (hg, rec)


# speedup vs baseline: 1.1423x; 1.1423x over previous
import jax
import jax.numpy as jnp
from jax import lax
from jax.experimental import pallas as pl
from jax.experimental.pallas import tpu as pltpu
from jax.experimental.pallas import tpu_sc as plsc

N = 10000
E = 320000
D = 128

NC = 2
NS = 16
NW = NC * NS
CH = 125
NCHUNK = 80
HC = NCHUNK // 2
NA = N
FR = 624
CR = 208
TAILF = N - NS * FR
TAILZ = NA - NS * FR


def _make_sc_agg(with_deg):
  mesh = plsc.VectorSubcoreMesh(core_axis_name="c", subcore_axis_name="s")
  out_type = [jax.ShapeDtypeStruct((NC, N, D), jnp.float32)]
  if with_deg:
    out_type.append(jax.ShapeDtypeStruct((NC, NA), jnp.float32))

  scratch = [
      pltpu.VMEM_SHARED((NA, D), jnp.float32),
      pltpu.VMEM((HC, CH), jnp.int32),
      pltpu.VMEM((HC, CH), jnp.int32),
      pltpu.VMEM((CH, D), jnp.float32),
      pltpu.VMEM((CH, D), jnp.float32),
      pltpu.SemaphoreType.DMA,
      pltpu.SemaphoreType.DMA,
      pltpu.SemaphoreType.DMA,
      pltpu.SemaphoreType.DMA,
  ]
  if with_deg:
    scratch += [
        pltpu.VMEM_SHARED((NA,), jnp.float32),
        pltpu.VMEM((128,), jnp.float32),
        pltpu.VMEM((FR,), jnp.float32),
        pltpu.SemaphoreType.DMA,
    ]

  def body(x_hbm, edges_hbm, z2d_hbm, *rest):
    if with_deg:
      (sums_hbm, deg_hbm, acc, src_v, dst_v, rows0, rows1,
       gsem0, gsem1, ssem0, ssem1, dacc, ones, dz, dsem) = rest
    else:
      (sums_hbm, acc, src_v, dst_v, rows0, rows1,
       gsem0, gsem1, ssem0, ssem1) = rest
    rows = (rows0, rows1)
    gsem = (gsem0, gsem1)
    ssem = (ssem0, ssem1)
    c = lax.axis_index("c")
    s = lax.axis_index("s")
    wid = c * NS + s

    def fire_gather(j, b):
      pltpu.async_copy(x_hbm.at[src_v.at[j]], rows[b], gsem[b])

    def stage_idx(half):
      pltpu.sync_copy(edges_hbm.at[0, wid, pl.ds(half * HC, HC)], src_v)
      pltpu.sync_copy(edges_hbm.at[1, wid, pl.ds(half * HC, HC)], dst_v)

    stage_idx(0)
    fire_gather(0, 0)

    for t in range(FR // CR):
      r0 = s * FR + t * CR
      pltpu.sync_copy(z2d_hbm.at[pl.ds(r0, CR), :], acc.at[pl.ds(r0, CR), :])
    @pl.when(s == NS - 1)
    def _():
      r0 = NS * FR
      pltpu.sync_copy(z2d_hbm.at[pl.ds(r0, TAILZ), :],
                      acc.at[pl.ds(r0, TAILZ), :])
    if with_deg:
      def zbody(i, _):
        dz[pl.ds(i * 16, 16)] = jnp.zeros((16,), jnp.float32)
        return 0
      lax.fori_loop(0, FR // 16, zbody, 0)
      pltpu.sync_copy(dz, dacc.at[pl.ds(s * FR, FR)])
      @pl.when(s == NS - 1)
      def _():
        pltpu.sync_copy(dz.at[pl.ds(0, TAILZ)],
                        dacc.at[pl.ds(NS * FR, TAILZ)])
      def obody(i, _):
        ones[pl.ds(i * 16, 16)] = jnp.ones((16,), jnp.float32)
        return 0
      lax.fori_loop(0, 8, obody, 0)
    plsc.subcore_barrier()

    for half in range(2):
      if half == 1:
        stage_idx(half)
        fire_gather(0, 0)

      def pair(p, _):
        for b in range(2):
          j = 2 * p + b
          pltpu.make_async_copy(x_hbm.at[src_v.at[j]], rows[b],
                                gsem[b]).wait()
          @pl.when(j >= 1)
          def _():
            pltpu.make_async_copy(rows[1 - b], acc.at[dst_v.at[j]],
                                  ssem[1 - b]).wait()
          @pl.when(j + 1 < HC)
          def _():
            fire_gather(j + 1, 1 - b)
          pltpu.async_copy(rows[b], acc.at[dst_v.at[j]], ssem[b], add=True)
          if with_deg:
            @pl.when(j >= 1)
            def _():
              pltpu.make_async_copy(ones.at[pl.ds(0, CH)],
                                    dacc.at[dst_v.at[j]], dsem).wait()
            pltpu.async_copy(ones.at[pl.ds(0, CH)], dacc.at[dst_v.at[j]],
                             dsem, add=True)
        return 0
      lax.fori_loop(0, HC // 2, pair, 0)

      pltpu.make_async_copy(rows[(HC - 1) % 2], acc.at[dst_v.at[0]],
                            ssem[(HC - 1) % 2]).wait()
      if with_deg:
        pltpu.make_async_copy(ones.at[pl.ds(0, CH)], dacc.at[dst_v.at[0]],
                              dsem).wait()

    plsc.subcore_barrier()

    for t in range(FR // CR):
      r0 = s * FR + t * CR
      pltpu.sync_copy(acc.at[pl.ds(r0, CR), :], sums_hbm.at[c, pl.ds(r0, CR), :])
    @pl.when(s == NS - 1)
    def _():
      r0 = NS * FR
      pltpu.sync_copy(acc.at[pl.ds(r0, TAILF), :],
                      sums_hbm.at[c, pl.ds(r0, TAILF), :])
    if with_deg:
      @pl.when(s == 0)
      def _():
        pltpu.sync_copy(dacc, deg_hbm.at[c])

  return pl.kernel(body, out_type=out_type, mesh=mesh, scratch_types=scratch)


_sc_agg_deg = _make_sc_agg(True)
_sc_agg = _make_sc_agg(False)


BN = 1000


def _tc1_body(x_ref, s_ref, d0_ref, d1_ref, w_ref, b_ref, o_ref):
  deg = jnp.maximum(d0_ref[...] + d1_ref[...], 1.0)
  agg = (s_ref[0] + s_ref[1]) / deg
  h = (x_ref[...] + agg) @ w_ref[...] + b_ref[...]
  o_ref[...] = jnp.maximum(h, 0.0)


def _tc1(x, sums, d0, d1, w, b):
  row = pl.BlockSpec((BN, D), lambda i: (i, 0))
  par = pl.BlockSpec((NC, BN, D), lambda i: (0, i, 0))
  col = pl.BlockSpec((BN, 1), lambda i: (i, 0))
  full = pl.BlockSpec((D, D), lambda i: (0, 0))
  bias = pl.BlockSpec((1, D), lambda i: (0, 0))
  return pl.pallas_call(
      _tc1_body,
      grid=(N // BN,),
      in_specs=[row, par, col, col, full, bias],
      out_specs=row,
      out_shape=jax.ShapeDtypeStruct((N, D), jnp.float32),
  )(x, sums, d0, d1, w, b)


def _tc2_body(h_ref, s_ref, d0_ref, d1_ref, w2_ref, b2_ref,
              wd1_ref, bd1_ref, wd2_ref, bd2_ref, hg_ref, rec_ref, acc_ref):
  i = pl.program_id(0)

  @pl.when(i == 0)
  def _():
    acc_ref[...] = jnp.zeros_like(acc_ref)

  deg = jnp.maximum(d0_ref[...] + d1_ref[...], 1.0)
  agg = (s_ref[0] + s_ref[1]) / deg
  h2 = jnp.maximum((h_ref[...] + agg) @ w2_ref[...] + b2_ref[...], 0.0)
  acc_ref[...] += jnp.sum(h2, axis=0, keepdims=True)

  @pl.when(i == pl.num_programs(0) - 1)
  def _():
    hg = acc_ref[...] * (1.0 / N)
    hg_ref[...] = hg
    r = jnp.maximum(hg @ wd1_ref[...] + bd1_ref[...], 0.0)
    rec_ref[...] = r @ wd2_ref[...] + bd2_ref[...]


def _tc2(h, sums, d0, d1, w2, b2, wd1, bd1, wd2, bd2):
  row = pl.BlockSpec((BN, D), lambda i: (i, 0))
  par = pl.BlockSpec((NC, BN, D), lambda i: (0, i, 0))
  col = pl.BlockSpec((BN, 1), lambda i: (i, 0))
  full = pl.BlockSpec((D, D), lambda i: (0, 0))
  bias = pl.BlockSpec((1, D), lambda i: (0, 0))
  out = pl.BlockSpec((1, D), lambda i: (0, 0))
  return pl.pallas_call(
      _tc2_body,
      grid=(N // BN,),
      in_specs=[row, par, col, col, full, bias, full, bias, full, bias],
      out_specs=[out, out],
      out_shape=[jax.ShapeDtypeStruct((1, D), jnp.float32),
                 jax.ShapeDtypeStruct((1, D), jnp.float32)],
      scratch_shapes=[pltpu.VMEM((1, D), jnp.float32)],
  )(h, sums, d0, d1, w2, b2, wd1, bd1, wd2, bd2)


@jax.jit
def kernel(features, edge_index, W1, b1, W2, b2, Wd1, bd1, Wd2, bd2):
  edges = edge_index.reshape(2, NW, NCHUNK, CH)
  z2d = jnp.zeros((NA, D), jnp.float32)

  sums1, deg = _sc_agg_deg(features, edges, z2d)
  d0 = deg[0, :N].reshape(N, 1)
  d1 = deg[1, :N].reshape(N, 1)
  h1 = _tc1(features, sums1, d0, d1, W1, b1.reshape(1, D))

  (sums2,) = _sc_agg(h1, edges, z2d)
  hg, rec = _tc2(h1, sums2, d0, d1, W2, b2.reshape(1, D),
                 Wd1, bd1.reshape(1, D), Wd2, bd2.reshape(1, D))
  return (hg, rec)


# speedup vs baseline: 1.1593x; 1.0149x over previous
import jax
import jax.numpy as jnp
from jax import lax
from jax.experimental import pallas as pl
from jax.experimental.pallas import tpu as pltpu
from jax.experimental.pallas import tpu_sc as plsc

N = 10000
E = 320000
D = 128

NC = 2
NS = 16
NW = NC * NS
CH = 125
NCHUNK = 80
HC = NCHUNK // 2
NA = N
FR = 624
CR = 208
TAILF = N - NS * FR
TAILZ = NA - NS * FR


def _make_sc_agg(with_deg):
  mesh = plsc.VectorSubcoreMesh(core_axis_name="c", subcore_axis_name="s")
  out_type = [jax.ShapeDtypeStruct((NC, N, D), jnp.float32)]
  if with_deg:
    out_type.append(jax.ShapeDtypeStruct((NC, NA), jnp.float32))

  scratch = [
      pltpu.VMEM_SHARED((NA, D), jnp.float32),
      pltpu.VMEM((HC, CH), jnp.int32),
      pltpu.VMEM((HC, CH), jnp.int32),
      pltpu.VMEM((CH, D), jnp.float32),
      pltpu.VMEM((CH, D), jnp.float32),
      pltpu.SemaphoreType.DMA,
      pltpu.SemaphoreType.DMA,
      pltpu.SemaphoreType.DMA,
      pltpu.SemaphoreType.DMA,
  ]
  if with_deg:
    scratch += [
        pltpu.VMEM_SHARED((NA,), jnp.float32),
        pltpu.VMEM((128,), jnp.float32),
        pltpu.VMEM((FR,), jnp.float32),
        pltpu.SemaphoreType.DMA,
    ]

  def body(x_hbm, edges_hbm, z2d_hbm, *rest):
    if with_deg:
      (sums_hbm, deg_hbm, acc, src_v, dst_v, rows0, rows1,
       gsem0, gsem1, ssem0, ssem1, dacc, ones, dz, dsem) = rest
    else:
      (sums_hbm, acc, src_v, dst_v, rows0, rows1,
       gsem0, gsem1, ssem0, ssem1) = rest
    rows = (rows0, rows1)
    gsem = (gsem0, gsem1)
    ssem = (ssem0, ssem1)
    c = lax.axis_index("c")
    s = lax.axis_index("s")
    wid = c * NS + s

    def fire_gather(j, b):
      pltpu.async_copy(x_hbm.at[src_v.at[j]], rows[b], gsem[b])

    def stage_idx(half):
      pltpu.sync_copy(edges_hbm.at[0, wid, pl.ds(half * HC, HC)], src_v)
      pltpu.sync_copy(edges_hbm.at[1, wid, pl.ds(half * HC, HC)], dst_v)

    stage_idx(0)
    fire_gather(0, 0)

    for t in range(FR // CR):
      r0 = s * FR + t * CR
      pltpu.sync_copy(z2d_hbm.at[pl.ds(r0, CR), :], acc.at[pl.ds(r0, CR), :])
    @pl.when(s == NS - 1)
    def _():
      r0 = NS * FR
      pltpu.sync_copy(z2d_hbm.at[pl.ds(r0, TAILZ), :],
                      acc.at[pl.ds(r0, TAILZ), :])
    if with_deg:
      def zbody(i, _):
        dz[pl.ds(i * 16, 16)] = jnp.zeros((16,), jnp.float32)
        return 0
      lax.fori_loop(0, FR // 16, zbody, 0)
      pltpu.sync_copy(dz, dacc.at[pl.ds(s * FR, FR)])
      @pl.when(s == NS - 1)
      def _():
        pltpu.sync_copy(dz.at[pl.ds(0, TAILZ)],
                        dacc.at[pl.ds(NS * FR, TAILZ)])
      def obody(i, _):
        ones[pl.ds(i * 16, 16)] = jnp.ones((16,), jnp.float32)
        return 0
      lax.fori_loop(0, 8, obody, 0)
    plsc.subcore_barrier()

    for half in range(2):
      if half == 1:
        stage_idx(half)
        fire_gather(0, 0)

      def pair(p, _):
        for b in range(2):
          j = 2 * p + b
          pltpu.make_async_copy(x_hbm.at[src_v.at[j]], rows[b],
                                gsem[b]).wait()
          @pl.when(j >= 1)
          def _():
            pltpu.make_async_copy(rows[1 - b], acc.at[dst_v.at[j]],
                                  ssem[1 - b]).wait()
          @pl.when(j + 1 < HC)
          def _():
            fire_gather(j + 1, 1 - b)
          pltpu.async_copy(rows[b], acc.at[dst_v.at[j]], ssem[b], add=True)
          if with_deg:
            @pl.when(j >= 1)
            def _():
              pltpu.make_async_copy(ones.at[pl.ds(0, CH)],
                                    dacc.at[dst_v.at[j]], dsem).wait()
            pltpu.async_copy(ones.at[pl.ds(0, CH)], dacc.at[dst_v.at[j]],
                             dsem, add=True)
        return 0
      lax.fori_loop(0, HC // 2, pair, 0)

      pltpu.make_async_copy(rows[(HC - 1) % 2], acc.at[dst_v.at[0]],
                            ssem[(HC - 1) % 2]).wait()
      if with_deg:
        pltpu.make_async_copy(ones.at[pl.ds(0, CH)], dacc.at[dst_v.at[0]],
                              dsem).wait()

    plsc.subcore_barrier()

    for t in range(FR // CR):
      r0 = s * FR + t * CR
      pltpu.sync_copy(acc.at[pl.ds(r0, CR), :], sums_hbm.at[c, pl.ds(r0, CR), :])
    @pl.when(s == NS - 1)
    def _():
      r0 = NS * FR
      pltpu.sync_copy(acc.at[pl.ds(r0, TAILF), :],
                      sums_hbm.at[c, pl.ds(r0, TAILF), :])
    if with_deg:
      @pl.when(s == 0)
      def _():
        pltpu.sync_copy(dacc, deg_hbm.at[c])

  return pl.kernel(body, out_type=out_type, mesh=mesh, scratch_types=scratch)


_sc_agg_deg = _make_sc_agg(True)
_sc_agg = _make_sc_agg(False)


BN = 2000


def _tc1_body(x_ref, s_ref, d0_ref, d1_ref, w_ref, b_ref, o_ref):
  deg = jnp.maximum(d0_ref[...] + d1_ref[...], 1.0)
  agg = (s_ref[0] + s_ref[1]) / deg
  h = (x_ref[...] + agg) @ w_ref[...] + b_ref[...]
  o_ref[...] = jnp.maximum(h, 0.0)


def _tc1(x, sums, d0, d1, w, b):
  row = pl.BlockSpec((BN, D), lambda i: (i, 0))
  par = pl.BlockSpec((NC, BN, D), lambda i: (0, i, 0))
  col = pl.BlockSpec((BN, 1), lambda i: (i, 0))
  full = pl.BlockSpec((D, D), lambda i: (0, 0))
  bias = pl.BlockSpec((1, D), lambda i: (0, 0))
  return pl.pallas_call(
      _tc1_body,
      grid=(N // BN,),
      in_specs=[row, par, col, col, full, bias],
      out_specs=row,
      out_shape=jax.ShapeDtypeStruct((N, D), jnp.float32),
  )(x, sums, d0, d1, w, b)


def _tc2_body(h_ref, s_ref, d0_ref, d1_ref, w2_ref, b2_ref,
              wd1_ref, bd1_ref, wd2_ref, bd2_ref, hg_ref, rec_ref, acc_ref):
  i = pl.program_id(0)

  @pl.when(i == 0)
  def _():
    acc_ref[...] = jnp.zeros_like(acc_ref)

  deg = jnp.maximum(d0_ref[...] + d1_ref[...], 1.0)
  agg = (s_ref[0] + s_ref[1]) / deg
  h2 = jnp.maximum((h_ref[...] + agg) @ w2_ref[...] + b2_ref[...], 0.0)
  acc_ref[...] += jnp.sum(h2, axis=0, keepdims=True)

  @pl.when(i == pl.num_programs(0) - 1)
  def _():
    hg = acc_ref[...] * (1.0 / N)
    hg_ref[...] = hg
    r = jnp.maximum(hg @ wd1_ref[...] + bd1_ref[...], 0.0)
    rec_ref[...] = r @ wd2_ref[...] + bd2_ref[...]


def _tc2(h, sums, d0, d1, w2, b2, wd1, bd1, wd2, bd2):
  row = pl.BlockSpec((BN, D), lambda i: (i, 0))
  par = pl.BlockSpec((NC, BN, D), lambda i: (0, i, 0))
  col = pl.BlockSpec((BN, 1), lambda i: (i, 0))
  full = pl.BlockSpec((D, D), lambda i: (0, 0))
  bias = pl.BlockSpec((1, D), lambda i: (0, 0))
  out = pl.BlockSpec((1, D), lambda i: (0, 0))
  return pl.pallas_call(
      _tc2_body,
      grid=(N // BN,),
      in_specs=[row, par, col, col, full, bias, full, bias, full, bias],
      out_specs=[out, out],
      out_shape=[jax.ShapeDtypeStruct((1, D), jnp.float32),
                 jax.ShapeDtypeStruct((1, D), jnp.float32)],
      scratch_shapes=[pltpu.VMEM((1, D), jnp.float32)],
  )(h, sums, d0, d1, w2, b2, wd1, bd1, wd2, bd2)


@jax.jit
def kernel(features, edge_index, W1, b1, W2, b2, Wd1, bd1, Wd2, bd2):
  edges = edge_index.reshape(2, NW, NCHUNK, CH)
  z2d = jnp.zeros((NA, D), jnp.float32)

  sums1, deg = _sc_agg_deg(features, edges, z2d)
  d0 = deg[0, :N].reshape(N, 1)
  d1 = deg[1, :N].reshape(N, 1)
  h1 = _tc1(features, sums1, d0, d1, W1, b1.reshape(1, D))

  (sums2,) = _sc_agg(h1, edges, z2d)
  hg, rec = _tc2(h1, sums2, d0, d1, W2, b2.reshape(1, D),
                 Wd1, bd1.reshape(1, D), Wd2, bd2.reshape(1, D))
  return (hg, rec)
